# baseline (speedup 1.0000x reference)
import jax
import jax.numpy as jnp
from jax import lax
from jax.experimental import pallas as pl
from jax.experimental.pallas import tpu as pltpu
from jax.experimental.pallas import tpu_sc as plsc

NF = 26
V = 100000
B = 16384
NC, NS, L = 2, 16, 16
NW = NC * NS
HB = B // 2
RPT = HB // NS
TCH = 128
GCH = 2048
BPW = B // NW

_CP = pltpu.CompilerParams(needs_layout_passes=False)
_MESH = dict(core_axis_name="c", subcore_axis_name="s",
             num_cores=NC, num_subcores=NS)


def _main_body(xflat_hbm, t2d_hbm, part_hbm,
               xT_sh, trow, bufA, bufB, idxc, valc, sem):
    c_id = lax.axis_index("c")
    s_id = lax.axis_index("s")
    wid = s_id * NC + c_id
    iota = lax.iota(jnp.int32, L)
    vec26 = iota * NF

    @pl.when(wid < NF)
    def _():
        pltpu.async_copy(t2d_hbm.at[wid], trow, sem)

    for h in range(2):
        def chunk_a(k, _):
            r0 = h * HB + s_id * RPT + k * TCH
            pltpu.sync_copy(xflat_hbm.at[pl.ds(r0 * NF, TCH * NF)], bufA)

            def vstep(v, _):
                f = v // (TCH // L)
                j = v % (TCH // L)
                p = vec26 + (j * (L * NF) + f)
                bufB[f, pl.ds(j * L, L)] = plsc.load_gather(bufA, [p])
                return 0
            lax.fori_loop(0, NF * (TCH // L), vstep, 0)
            pltpu.sync_copy(bufB, xT_sh.at[:, pl.ds(s_id * RPT + k * TCH, TCH)])
            return 0
        lax.fori_loop(0, RPT // TCH, chunk_a, 0)
        plsc.subcore_barrier()

        @pl.when(wid < NF)
        def _():
            if h == 0:
                pltpu.make_async_copy(t2d_hbm.at[wid], trow, sem).wait()

            def chunk_b(cb, _):
                lbase = cb * GCH
                pltpu.sync_copy(xT_sh.at[wid, pl.ds(lbase, GCH)], idxc)

                def gvec(i, _):
                    ids = idxc[pl.ds(i * L, L)]
                    valc[pl.ds(i * L, L)] = plsc.load_gather(trow, [ids])
                    return 0
                lax.fori_loop(0, GCH // L, gvec, 0)
                pltpu.sync_copy(valc, part_hbm.at[wid, pl.ds(h * HB + lbase, GCH)])
                return 0
            lax.fori_loop(0, HB // GCH, chunk_b, 0)
        plsc.subcore_barrier()


def _reduce_body(part_hbm, out_hbm, pv, out_v):
    wid = lax.axis_index("s") * NC + lax.axis_index("c")
    pltpu.sync_copy(part_hbm.at[:, pl.ds(wid * BPW, BPW)], pv)

    def red_chunk(c, _):
        acc = jnp.zeros((L,), jnp.float32)
        for f in range(NF):
            acc = acc + pv[f, pl.ds(c * L, L)]
        out_v[pl.ds(c * L, L)] = acc
        return 0
    lax.fori_loop(0, BPW // L, red_chunk, 0)
    pltpu.sync_copy(out_v, out_hbm.at[pl.ds(wid * BPW, BPW)])


def kernel(X, table):
    xflat = X.reshape(B * NF)
    t2d = table.reshape(NF, V)

    partials = pl.kernel(
        _main_body,
        out_type=jax.ShapeDtypeStruct((NF, B), jnp.float32),
        mesh=plsc.VectorSubcoreMesh(**_MESH),
        scratch_types=[
            pltpu.VMEM_SHARED((32, HB), jnp.int32),
            pltpu.VMEM((V,), jnp.float32),
            pltpu.VMEM((TCH * NF,), jnp.int32),
            pltpu.VMEM((32, TCH), jnp.int32),
            pltpu.VMEM((GCH,), jnp.int32),
            pltpu.VMEM((GCH,), jnp.float32),
            pltpu.SemaphoreType.DMA,
        ],
        compiler_params=_CP,
    )(xflat, t2d)

    out = pl.kernel(
        _reduce_body,
        out_type=jax.ShapeDtypeStruct((B,), jnp.float32),
        mesh=plsc.VectorSubcoreMesh(**_MESH),
        scratch_types=[
            pltpu.VMEM((NF, BPW), jnp.float32),
            pltpu.VMEM((BPW,), jnp.float32),
        ],
        compiler_params=_CP,
    )(partials)
    return out.reshape(B, 1)


# speedup vs baseline: 1.0278x; 1.0278x over previous
import jax
import jax.numpy as jnp
from jax import lax
from jax.experimental import pallas as pl
from jax.experimental.pallas import tpu as pltpu
from jax.experimental.pallas import tpu_sc as plsc

NF = 26
V = 100000
B = 16384
NC, NS, L = 2, 16, 16
NW = NC * NS
HB = B // 2
RPT = HB // NS
TCH = 128
GCH = 2048
BPW = B // NW

_CP = pltpu.CompilerParams(needs_layout_passes=False)
_MESH = dict(core_axis_name="c", subcore_axis_name="s",
             num_cores=NC, num_subcores=NS)


def _main_body(xflat_hbm, t2d_hbm, part_hbm,
               xT_sh, trow, bufA, bufB, idxc, valc, sem):
    c_id = lax.axis_index("c")
    s_id = lax.axis_index("s")
    wid = s_id * NC + c_id
    iota = lax.iota(jnp.int32, L)
    vec26 = iota * NF

    @pl.when(wid < NF)
    def _():
        pltpu.async_copy(t2d_hbm.at[wid], trow, sem)

    for h in range(2):
        def chunk_a(k, _):
            r0 = h * HB + s_id * RPT + k * TCH
            pltpu.sync_copy(xflat_hbm.at[pl.ds(r0 * NF, TCH * NF)], bufA)

            def frow(f, _):
                for j in range(TCH // L):
                    p = vec26 + (j * (L * NF) + f)
                    bufB[f, pl.ds(j * L, L)] = plsc.load_gather(bufA, [p])
                return 0
            lax.fori_loop(0, NF, frow, 0)
            pltpu.sync_copy(bufB, xT_sh.at[:, pl.ds(s_id * RPT + k * TCH, TCH)])
            return 0
        lax.fori_loop(0, RPT // TCH, chunk_a, 0)
        plsc.subcore_barrier()

        @pl.when(wid < NF)
        def _():
            if h == 0:
                pltpu.make_async_copy(t2d_hbm.at[wid], trow, sem).wait()

            def chunk_b(cb, _):
                lbase = cb * GCH
                pltpu.sync_copy(xT_sh.at[wid, pl.ds(lbase, GCH)], idxc)

                def g8(i, _):
                    for j in range(8):
                        off = i * (8 * L) + j * L
                        ids = idxc[pl.ds(off, L)]
                        valc[pl.ds(off, L)] = plsc.load_gather(trow, [ids])
                    return 0
                lax.fori_loop(0, GCH // (8 * L), g8, 0)
                pltpu.sync_copy(valc, part_hbm.at[wid, pl.ds(h * HB + lbase, GCH)])
                return 0
            lax.fori_loop(0, HB // GCH, chunk_b, 0)
        plsc.subcore_barrier()


def _reduce_body(part_hbm, out_hbm, pv, out_v):
    wid = lax.axis_index("s") * NC + lax.axis_index("c")
    pltpu.sync_copy(part_hbm.at[:, pl.ds(wid * BPW, BPW)], pv)

    def red_chunk(c, _):
        acc = jnp.zeros((L,), jnp.float32)
        for f in range(NF):
            acc = acc + pv[f, pl.ds(c * L, L)]
        out_v[pl.ds(c * L, L)] = acc
        return 0
    lax.fori_loop(0, BPW // L, red_chunk, 0)
    pltpu.sync_copy(out_v, out_hbm.at[pl.ds(wid * BPW, BPW)])


def kernel(X, table):
    xflat = X.reshape(B * NF)
    t2d = table.reshape(NF, V)

    partials = pl.kernel(
        _main_body,
        out_type=jax.ShapeDtypeStruct((NF, B), jnp.float32),
        mesh=plsc.VectorSubcoreMesh(**_MESH),
        scratch_types=[
            pltpu.VMEM_SHARED((32, HB), jnp.int32),
            pltpu.VMEM((V,), jnp.float32),
            pltpu.VMEM((TCH * NF,), jnp.int32),
            pltpu.VMEM((32, TCH), jnp.int32),
            pltpu.VMEM((GCH,), jnp.int32),
            pltpu.VMEM((GCH,), jnp.float32),
            pltpu.SemaphoreType.DMA,
        ],
        compiler_params=_CP,
    )(xflat, t2d)

    out = pl.kernel(
        _reduce_body,
        out_type=jax.ShapeDtypeStruct((B,), jnp.float32),
        mesh=plsc.VectorSubcoreMesh(**_MESH),
        scratch_types=[
            pltpu.VMEM((NF, BPW), jnp.float32),
            pltpu.VMEM((BPW,), jnp.float32),
        ],
        compiler_params=_CP,
    )(partials)
    return out.reshape(B, 1)
